# initial kernel scaffold (unmeasured)
import jax
import jax.numpy as jnp
from jax import lax
from jax.experimental import pallas as pl
from jax.experimental.pallas import tpu as pltpu

N_DEV = 4
H_CHUNK = 512


def kernel(x, Wg, Wu, Wd):
    m, k = x.shape
    h_per = Wg.shape[1]
    steps = h_per // H_CHUNK
    ch = m // N_DEV

    def body(x_ref, wg_ref, wu_ref, wd_ref, out_ref,
             xbf_ref, acc_ref, comm_ref, send_sems, recv_sems):
        j = pl.program_id(0)

        @pl.when(j == 0)
        def _init():
            xbf_ref[...] = x_ref[...].astype(jnp.bfloat16)
            acc_ref[...] = jnp.zeros_like(acc_ref)

        xb = xbf_ref[...]
        wg = wg_ref[...].astype(jnp.bfloat16)
        wu = wu_ref[...].astype(jnp.bfloat16)
        wd = wd_ref[...].astype(jnp.bfloat16)
        g = jnp.dot(xb, wg, preferred_element_type=jnp.float32)
        u = jnp.dot(xb, wu, preferred_element_type=jnp.float32)
        hh = (g * u * jax.nn.sigmoid(u)).astype(jnp.bfloat16)
        acc_ref[...] += jnp.dot(hh, wd, preferred_element_type=jnp.float32)

        @pl.when(j == steps - 1)
        def _ring():
            d = lax.axis_index("i")
            right = (d + 1) % N_DEV
            left = (d + N_DEV - 1) % N_DEV

            barrier = pltpu.get_barrier_semaphore()
            for nbr in (left, right):
                pl.semaphore_signal(
                    barrier, inc=1,
                    device_id=(nbr,), device_id_type=pl.DeviceIdType.MESH,
                )
            pl.semaphore_wait(barrier, 2)

            c0 = (d + N_DEV - 1) % N_DEV
            comm_ref[0, :, :] = acc_ref[pl.ds(c0 * ch, ch), :].astype(jnp.bfloat16)

            red = None
            for s in range(N_DEV - 1):
                rdma = pltpu.make_async_remote_copy(
                    src_ref=comm_ref.at[s],
                    dst_ref=comm_ref.at[3 + s],
                    send_sem=send_sems.at[s],
                    recv_sem=recv_sems.at[s],
                    device_id=(right,),
                    device_id_type=pl.DeviceIdType.MESH,
                )
                rdma.start()
                rdma.wait()
                c = (d + N_DEV - 2 - s) % N_DEV
                red = (comm_ref[3 + s].astype(jnp.float32)
                       + acc_ref[pl.ds(c * ch, ch), :])
                if s < N_DEV - 2:
                    comm_ref[s + 1, :, :] = red.astype(jnp.bfloat16)

            out_ref[pl.ds(d * ch, ch), :] = red
            comm_ref[6, :, :] = red.astype(jnp.bfloat16)

            for s in range(N_DEV - 1):
                rdma = pltpu.make_async_remote_copy(
                    src_ref=comm_ref.at[6 + s],
                    dst_ref=comm_ref.at[7 + s],
                    send_sem=send_sems.at[3 + s],
                    recv_sem=recv_sems.at[3 + s],
                    device_id=(right,),
                    device_id_type=pl.DeviceIdType.MESH,
                )
                rdma.start()
                rdma.wait()
                c = (d + N_DEV - 1 - s) % N_DEV
                out_ref[pl.ds(c * ch, ch), :] = comm_ref[7 + s].astype(jnp.float32)

    return pl.pallas_call(
        body,
        grid=(steps,),
        in_specs=[
            pl.BlockSpec((m, k), lambda j: (0, 0)),
            pl.BlockSpec((k, H_CHUNK), lambda j: (0, j)),
            pl.BlockSpec((k, H_CHUNK), lambda j: (0, j)),
            pl.BlockSpec((H_CHUNK, m), lambda j: (j, 0)),
        ],
        out_specs=pl.BlockSpec((m, m), lambda j: (0, 0)),
        out_shape=jax.ShapeDtypeStruct((m, m), jnp.float32),
        scratch_shapes=[
            pltpu.VMEM((m, k), jnp.bfloat16),
            pltpu.VMEM((m, m), jnp.float32),
            pltpu.VMEM((10, ch, m), jnp.bfloat16),
            pltpu.SemaphoreType.DMA((6,)),
            pltpu.SemaphoreType.DMA((6,)),
        ],
        compiler_params=pltpu.CompilerParams(
            collective_id=0,
            dimension_semantics=("arbitrary",),
        ),
    )(x, Wg, Wu, Wd)


# baseline (device time: 154534 ns/iter reference)
import jax
import jax.numpy as jnp
from jax import lax
from jax.experimental import pallas as pl
from jax.experimental.pallas import tpu as pltpu

N_DEV = 4
H_CHUNK = 256


def kernel(x, Wg, Wu, Wd):
    m, k = x.shape
    h_per = Wg.shape[1]
    steps = h_per // H_CHUNK
    ch = m // N_DEV

    def body(x_ref, wg_ref, wu_ref, wd_ref, out_ref,
             xbf_ref, acc_ref, comm_ref, send_sems, recv_sems):
        j = pl.program_id(0)

        @pl.when(j == 0)
        def _init():
            xbf_ref[...] = x_ref[...].astype(jnp.bfloat16)
            acc_ref[...] = jnp.zeros_like(acc_ref)

        xb = xbf_ref[...]
        wg = wg_ref[...].astype(jnp.bfloat16)
        wu = wu_ref[...].astype(jnp.bfloat16)
        wd = wd_ref[...].astype(jnp.bfloat16)
        g = jnp.dot(xb, wg, preferred_element_type=jnp.float32)
        u = jnp.dot(xb, wu, preferred_element_type=jnp.float32)
        hh = (g * u * jax.nn.sigmoid(u)).astype(jnp.bfloat16)
        acc_ref[...] += jnp.dot(hh, wd, preferred_element_type=jnp.float32)

        @pl.when(j == steps - 1)
        def _ring():
            d = lax.axis_index("i")
            right = (d + 1) % N_DEV
            left = (d + N_DEV - 1) % N_DEV

            barrier = pltpu.get_barrier_semaphore()
            for nbr in (left, right):
                pl.semaphore_signal(
                    barrier, inc=1,
                    device_id=(nbr,), device_id_type=pl.DeviceIdType.MESH,
                )
            pl.semaphore_wait(barrier, 2)

            c0 = (d + N_DEV - 1) % N_DEV
            comm_ref[0, :, :] = acc_ref[pl.ds(c0 * ch, ch), :].astype(jnp.bfloat16)

            red = None
            for s in range(N_DEV - 1):
                rdma = pltpu.make_async_remote_copy(
                    src_ref=comm_ref.at[0],
                    dst_ref=comm_ref.at[1 + s],
                    send_sem=send_sems.at[s],
                    recv_sem=recv_sems.at[s],
                    device_id=(right,),
                    device_id_type=pl.DeviceIdType.MESH,
                )
                rdma.start()
                rdma.wait()
                c = (d + N_DEV - 2 - s) % N_DEV
                red = (comm_ref[1 + s].astype(jnp.float32)
                       + acc_ref[pl.ds(c * ch, ch), :])
                if s < N_DEV - 2:
                    comm_ref[0, :, :] = red.astype(jnp.bfloat16)

            out_ref[pl.ds(d * ch, ch), :] = red.astype(jnp.bfloat16)
            comm_ref[4, :, :] = red.astype(jnp.bfloat16)

            for s in range(N_DEV - 1):
                rdma = pltpu.make_async_remote_copy(
                    src_ref=comm_ref.at[4 + s],
                    dst_ref=comm_ref.at[5 + s],
                    send_sem=send_sems.at[3 + s],
                    recv_sem=recv_sems.at[3 + s],
                    device_id=(right,),
                    device_id_type=pl.DeviceIdType.MESH,
                )
                rdma.start()
                rdma.wait()
                c = (d + N_DEV - 1 - s) % N_DEV
                out_ref[pl.ds(c * ch, ch), :] = comm_ref[5 + s]

    return pl.pallas_call(
        body,
        grid=(steps,),
        in_specs=[
            pl.BlockSpec((m, k), lambda j: (0, 0)),
            pl.BlockSpec((k, H_CHUNK), lambda j: (0, j)),
            pl.BlockSpec((k, H_CHUNK), lambda j: (0, j)),
            pl.BlockSpec((H_CHUNK, m), lambda j: (j, 0)),
        ],
        out_specs=pl.BlockSpec((m, m), lambda j: (0, 0)),
        out_shape=jax.ShapeDtypeStruct((m, m), jnp.bfloat16),
        scratch_shapes=[
            pltpu.VMEM((m, k), jnp.bfloat16),
            pltpu.VMEM((m, m), jnp.float32),
            pltpu.VMEM((8, ch, m), jnp.bfloat16),
            pltpu.SemaphoreType.DMA((6,)),
            pltpu.SemaphoreType.DMA((6,)),
        ],
        compiler_params=pltpu.CompilerParams(
            collective_id=0,
            dimension_semantics=("arbitrary",),
            vmem_limit_bytes=100 * 1024 * 1024,
        ),
    )(x, Wg, Wu, Wd)


# device time: 116615 ns/iter; 1.3252x vs baseline; 1.3252x over previous
import jax
import jax.numpy as jnp
from jax import lax
from jax.experimental import pallas as pl
from jax.experimental.pallas import tpu as pltpu

N_DEV = 4
H_CHUNK = 256


def kernel(x, Wg, Wu, Wd):
    m, k = x.shape
    h_per = Wg.shape[1]
    steps = h_per // H_CHUNK
    ch = m // N_DEV

    def body(x_ref, wg_ref, wu_ref, wd_ref, out_ref,
             xbf_ref, acc_ref, comm_ref, send_sems, recv_sems):
        j = pl.program_id(0)

        @pl.when(j == 0)
        def _init():
            xbf_ref[...] = x_ref[...].astype(jnp.bfloat16)
            acc_ref[...] = jnp.zeros_like(acc_ref)

        xb = xbf_ref[...]
        wg = wg_ref[...].astype(jnp.bfloat16)
        wu = wu_ref[...].astype(jnp.bfloat16)
        wd = wd_ref[...].astype(jnp.bfloat16)
        g = jnp.dot(xb, wg, preferred_element_type=jnp.float32)
        u = jnp.dot(xb, wu, preferred_element_type=jnp.float32)
        hh = (g * u * jax.nn.sigmoid(u)).astype(jnp.bfloat16)
        acc_ref[...] += jnp.dot(hh, wd, preferred_element_type=jnp.float32)

        @pl.when(j == steps - 1)
        def _ring():
            d = lax.axis_index("i")
            right = (d + 1) % N_DEV
            left = (d + N_DEV - 1) % N_DEV
            h2 = ch // 2

            barrier = pltpu.get_barrier_semaphore()
            for nbr in (left, right):
                pl.semaphore_signal(
                    barrier, inc=1,
                    device_id=(nbr,), device_id_type=pl.DeviceIdType.MESH,
                )
            pl.semaphore_wait(barrier, 2)

            c_cw = (d + N_DEV - 1) % N_DEV
            c_ccw = (d + 1) % N_DEV
            comm_ref[0, :, :] = acc_ref[pl.ds(c_cw * ch, h2), :].astype(jnp.bfloat16)
            comm_ref[8, :, :] = acc_ref[pl.ds(c_ccw * ch + h2, h2), :].astype(
                jnp.bfloat16)

            red_cw = red_ccw = None
            for s in range(N_DEV - 1):
                rd_cw = pltpu.make_async_remote_copy(
                    src_ref=comm_ref.at[0],
                    dst_ref=comm_ref.at[1 + s],
                    send_sem=send_sems.at[s],
                    recv_sem=recv_sems.at[s],
                    device_id=(right,),
                    device_id_type=pl.DeviceIdType.MESH,
                )
                rd_ccw = pltpu.make_async_remote_copy(
                    src_ref=comm_ref.at[8],
                    dst_ref=comm_ref.at[9 + s],
                    send_sem=send_sems.at[6 + s],
                    recv_sem=recv_sems.at[6 + s],
                    device_id=(left,),
                    device_id_type=pl.DeviceIdType.MESH,
                )
                rd_cw.start()
                rd_ccw.start()
                rd_cw.wait()
                rd_ccw.wait()
                c_cw = (d + N_DEV - 2 - s) % N_DEV
                c_ccw = (d + 2 + s) % N_DEV
                red_cw = (comm_ref[1 + s].astype(jnp.float32)
                          + acc_ref[pl.ds(c_cw * ch, h2), :])
                red_ccw = (comm_ref[9 + s].astype(jnp.float32)
                           + acc_ref[pl.ds(c_ccw * ch + h2, h2), :])
                if s < N_DEV - 2:
                    comm_ref[0, :, :] = red_cw.astype(jnp.bfloat16)
                    comm_ref[8, :, :] = red_ccw.astype(jnp.bfloat16)

            out_ref[pl.ds(d * ch, h2), :] = red_cw.astype(jnp.bfloat16)
            out_ref[pl.ds(d * ch + h2, h2), :] = red_ccw.astype(jnp.bfloat16)
            comm_ref[4, :, :] = red_cw.astype(jnp.bfloat16)
            comm_ref[12, :, :] = red_ccw.astype(jnp.bfloat16)

            for s in range(N_DEV - 1):
                rd_cw = pltpu.make_async_remote_copy(
                    src_ref=comm_ref.at[4 + s],
                    dst_ref=comm_ref.at[5 + s],
                    send_sem=send_sems.at[3 + s],
                    recv_sem=recv_sems.at[3 + s],
                    device_id=(right,),
                    device_id_type=pl.DeviceIdType.MESH,
                )
                rd_ccw = pltpu.make_async_remote_copy(
                    src_ref=comm_ref.at[12 + s],
                    dst_ref=comm_ref.at[13 + s],
                    send_sem=send_sems.at[9 + s],
                    recv_sem=recv_sems.at[9 + s],
                    device_id=(left,),
                    device_id_type=pl.DeviceIdType.MESH,
                )
                rd_cw.start()
                rd_ccw.start()
                rd_cw.wait()
                rd_ccw.wait()
                c_cw = (d + N_DEV - 1 - s) % N_DEV
                c_ccw = (d + 1 + s) % N_DEV
                out_ref[pl.ds(c_cw * ch, h2), :] = comm_ref[5 + s]
                out_ref[pl.ds(c_ccw * ch + h2, h2), :] = comm_ref[13 + s]

    return pl.pallas_call(
        body,
        grid=(steps,),
        in_specs=[
            pl.BlockSpec((m, k), lambda j: (0, 0)),
            pl.BlockSpec((k, H_CHUNK), lambda j: (0, j)),
            pl.BlockSpec((k, H_CHUNK), lambda j: (0, j)),
            pl.BlockSpec((H_CHUNK, m), lambda j: (j, 0)),
        ],
        out_specs=pl.BlockSpec((m, m), lambda j: (0, 0)),
        out_shape=jax.ShapeDtypeStruct((m, m), jnp.bfloat16),
        scratch_shapes=[
            pltpu.VMEM((m, k), jnp.bfloat16),
            pltpu.VMEM((m, m), jnp.float32),
            pltpu.VMEM((16, ch // 2, m), jnp.bfloat16),
            pltpu.SemaphoreType.DMA((12,)),
            pltpu.SemaphoreType.DMA((12,)),
        ],
        compiler_params=pltpu.CompilerParams(
            collective_id=0,
            dimension_semantics=("arbitrary",),
            vmem_limit_bytes=100 * 1024 * 1024,
        ),
    )(x, Wg, Wu, Wd)


# device time: 116462 ns/iter; 1.3269x vs baseline; 1.0013x over previous
import jax
import jax.numpy as jnp
from jax import lax
from jax.experimental import pallas as pl
from jax.experimental.pallas import tpu as pltpu

N_DEV = 4
H_CHUNK = 256


def kernel(x, Wg, Wu, Wd):
    m, k = x.shape
    h_per = Wg.shape[1]
    steps = h_per // H_CHUNK
    ch = m // N_DEV

    def body(x_ref, wg_ref, wu_ref, wd_ref, out_ref,
             xbf_ref, acc_ref, comm_ref, send_sems, recv_sems):
        j = pl.program_id(0)

        @pl.when(j == 0)
        def _init():
            xbf_ref[...] = x_ref[...].astype(jnp.bfloat16)
            acc_ref[...] = jnp.zeros_like(acc_ref)

        xb = xbf_ref[...]
        wg = wg_ref[...].astype(jnp.bfloat16)
        wu = wu_ref[...].astype(jnp.bfloat16)
        wd = wd_ref[...].astype(jnp.bfloat16)
        g = jnp.dot(xb, wg, preferred_element_type=jnp.float32).astype(jnp.bfloat16)
        u = jnp.dot(xb, wu, preferred_element_type=jnp.float32).astype(jnp.bfloat16)
        hh = g * u * jax.nn.sigmoid(u)
        acc_ref[...] += jnp.dot(hh, wd, preferred_element_type=jnp.float32)

        @pl.when(j == steps - 1)
        def _ring():
            d = lax.axis_index("i")
            right = (d + 1) % N_DEV
            left = (d + N_DEV - 1) % N_DEV
            h2 = ch // 2

            barrier = pltpu.get_barrier_semaphore()
            for nbr in (left, right):
                pl.semaphore_signal(
                    barrier, inc=1,
                    device_id=(nbr,), device_id_type=pl.DeviceIdType.MESH,
                )
            pl.semaphore_wait(barrier, 2)

            c_cw = (d + N_DEV - 1) % N_DEV
            c_ccw = (d + 1) % N_DEV
            comm_ref[0, :, :] = acc_ref[pl.ds(c_cw * ch, h2), :].astype(jnp.bfloat16)
            comm_ref[8, :, :] = acc_ref[pl.ds(c_ccw * ch + h2, h2), :].astype(
                jnp.bfloat16)

            red_cw = red_ccw = None
            for s in range(N_DEV - 1):
                rd_cw = pltpu.make_async_remote_copy(
                    src_ref=comm_ref.at[0],
                    dst_ref=comm_ref.at[1 + s],
                    send_sem=send_sems.at[s],
                    recv_sem=recv_sems.at[s],
                    device_id=(right,),
                    device_id_type=pl.DeviceIdType.MESH,
                )
                rd_ccw = pltpu.make_async_remote_copy(
                    src_ref=comm_ref.at[8],
                    dst_ref=comm_ref.at[9 + s],
                    send_sem=send_sems.at[6 + s],
                    recv_sem=recv_sems.at[6 + s],
                    device_id=(left,),
                    device_id_type=pl.DeviceIdType.MESH,
                )
                rd_cw.start()
                rd_ccw.start()
                rd_cw.wait()
                rd_ccw.wait()
                c_cw = (d + N_DEV - 2 - s) % N_DEV
                c_ccw = (d + 2 + s) % N_DEV
                red_cw = (comm_ref[1 + s].astype(jnp.float32)
                          + acc_ref[pl.ds(c_cw * ch, h2), :])
                red_ccw = (comm_ref[9 + s].astype(jnp.float32)
                           + acc_ref[pl.ds(c_ccw * ch + h2, h2), :])
                if s < N_DEV - 2:
                    comm_ref[0, :, :] = red_cw.astype(jnp.bfloat16)
                    comm_ref[8, :, :] = red_ccw.astype(jnp.bfloat16)

            out_ref[pl.ds(d * ch, h2), :] = red_cw.astype(jnp.bfloat16)
            out_ref[pl.ds(d * ch + h2, h2), :] = red_ccw.astype(jnp.bfloat16)
            comm_ref[4, :, :] = red_cw.astype(jnp.bfloat16)
            comm_ref[12, :, :] = red_ccw.astype(jnp.bfloat16)

            for s in range(N_DEV - 1):
                rd_cw = pltpu.make_async_remote_copy(
                    src_ref=comm_ref.at[4 + s],
                    dst_ref=comm_ref.at[5 + s],
                    send_sem=send_sems.at[3 + s],
                    recv_sem=recv_sems.at[3 + s],
                    device_id=(right,),
                    device_id_type=pl.DeviceIdType.MESH,
                )
                rd_ccw = pltpu.make_async_remote_copy(
                    src_ref=comm_ref.at[12 + s],
                    dst_ref=comm_ref.at[13 + s],
                    send_sem=send_sems.at[9 + s],
                    recv_sem=recv_sems.at[9 + s],
                    device_id=(left,),
                    device_id_type=pl.DeviceIdType.MESH,
                )
                rd_cw.start()
                rd_ccw.start()
                rd_cw.wait()
                rd_ccw.wait()
                c_cw = (d + N_DEV - 1 - s) % N_DEV
                c_ccw = (d + 1 + s) % N_DEV
                out_ref[pl.ds(c_cw * ch, h2), :] = comm_ref[5 + s]
                out_ref[pl.ds(c_ccw * ch + h2, h2), :] = comm_ref[13 + s]

    return pl.pallas_call(
        body,
        grid=(steps,),
        in_specs=[
            pl.BlockSpec((m, k), lambda j: (0, 0)),
            pl.BlockSpec((k, H_CHUNK), lambda j: (0, j)),
            pl.BlockSpec((k, H_CHUNK), lambda j: (0, j)),
            pl.BlockSpec((H_CHUNK, m), lambda j: (j, 0)),
        ],
        out_specs=pl.BlockSpec((m, m), lambda j: (0, 0)),
        out_shape=jax.ShapeDtypeStruct((m, m), jnp.bfloat16),
        scratch_shapes=[
            pltpu.VMEM((m, k), jnp.bfloat16),
            pltpu.VMEM((m, m), jnp.float32),
            pltpu.VMEM((16, ch // 2, m), jnp.bfloat16),
            pltpu.SemaphoreType.DMA((12,)),
            pltpu.SemaphoreType.DMA((12,)),
        ],
        compiler_params=pltpu.CompilerParams(
            collective_id=0,
            dimension_semantics=("arbitrary",),
            vmem_limit_bytes=100 * 1024 * 1024,
        ),
    )(x, Wg, Wu, Wd)
